# Initial kernel scaffold; baseline (speedup 1.0000x reference)
import jax
import jax.numpy as jnp
from jax.experimental import pallas as pl


def kernel(graph_x, edge_index, edge_attr, velocity_sequence_noise, enc_node, enc_edge, mp_edge, mp_node, dec, norm_stats):
    raise NotImplementedError("write your pallas kernel here")


# speedup vs baseline: 2.8359x; 2.8359x over previous
import functools

import jax
import jax.numpy as jnp
from jax import lax
from jax.experimental import pallas as pl
from jax.experimental.pallas import tpu as pltpu
from jax.experimental.pallas import tpu_sc as plsc

_N = 10000
_E = 160000
_H = 128
_MP = 3

_NC = 2
_NS = 16
_NW = _NC * _NS

_CH = 128
_NCHK = _E // _CH
_JFULL = _NCHK // _NW
_REM = _NCHK - _JFULL * _NW
_NPT = 624
_NREM = _N - _NS * _NPT

_NB = 1000
_EB = 1000

@functools.cache
def _sc_mesh():
    return plsc.VectorSubcoreMesh(
        core_axis_name="c", subcore_axis_name="s",
        num_cores=_NC, num_subcores=_NS)


def _f32dot(a, b):
    return jnp.dot(a, b, preferred_element_type=jnp.float32)


def _ln(h, g, b):
    mu = jnp.mean(h, axis=-1, keepdims=True)
    var = jnp.mean((h - mu) ** 2, axis=-1, keepdims=True)
    return (h - mu) / jnp.sqrt(var + 1e-5) * g + b


def _fullspec(shape):
    n = len(shape)
    return pl.BlockSpec(shape, lambda i, _n=n: (0,) * _n)


def _rowspec(blk, d):
    return pl.BlockSpec((blk, d), lambda i: (i, 0))


def _enc_node_body(x_ref, w1_ref, b1_ref, w2_ref, b2_ref, w3_ref, b3_ref,
                   g_ref, be_ref, o_ref):
    x = x_ref[...]
    t = x[:, 0:1].astype(jnp.int32)
    oh = (lax.broadcasted_iota(jnp.int32, (_NB, 9), 1) == t).astype(jnp.float32)
    feats = jnp.concatenate([x[:, 1:3], oh], axis=-1)
    h = jnp.maximum(_f32dot(feats, w1_ref[...]) + b1_ref[...], 0.0)
    h = jnp.maximum(_f32dot(h, w2_ref[...]) + b2_ref[...], 0.0)
    h = _f32dot(h, w3_ref[...]) + b3_ref[...]
    o_ref[...] = _ln(h, g_ref[...], be_ref[...])


def _enc_edge_body(x_ref, w1_ref, b1_ref, w2_ref, b2_ref, w3_ref, b3_ref,
                   g_ref, be_ref, o_ref):
    h = jnp.maximum(_f32dot(x_ref[...], w1_ref[...]) + b1_ref[...], 0.0)
    h = jnp.maximum(_f32dot(h, w2_ref[...]) + b2_ref[...], 0.0)
    h = _f32dot(h, w3_ref[...]) + b3_ref[...]
    o_ref[...] = _ln(h, g_ref[...], be_ref[...])


def _pq_body(hv_ref, ws_ref, wd_ref, p_ref, q_ref):
    hv = hv_ref[...]
    p_ref[...] = _f32dot(hv, ws_ref[...])
    q_ref[...] = _f32dot(hv, wd_ref[...])


def _edge_mlp_body(he_ref, r_ref, w1_ref, b1_ref, w2_ref, b2_ref,
                   w3_ref, b3_ref, g_ref, be_ref, o_ref):
    he = he_ref[...]
    h = jnp.maximum(_f32dot(he, w1_ref[...]) + r_ref[...] + b1_ref[...], 0.0)
    h = jnp.maximum(_f32dot(h, w2_ref[...]) + b2_ref[...], 0.0)
    h = _f32dot(h, w3_ref[...]) + b3_ref[...]
    o_ref[...] = _ln(h, g_ref[...], be_ref[...]) + he


def _node_mlp_body(hv_ref, part_ref, wv_ref, wa_ref, b1_ref, w2_ref, b2_ref,
                   w3_ref, b3_ref, g_ref, be_ref, o_ref):
    hv = hv_ref[...]
    agg = part_ref[0] + part_ref[1]
    h = jnp.maximum(_f32dot(hv, wv_ref[...]) + _f32dot(agg, wa_ref[...])
                    + b1_ref[...], 0.0)
    h = jnp.maximum(_f32dot(h, w2_ref[...]) + b2_ref[...], 0.0)
    h = _f32dot(h, w3_ref[...]) + b3_ref[...]
    o_ref[...] = _ln(h, g_ref[...], be_ref[...]) + hv


def _dec_body(hv_ref, fr_ref, w1_ref, b1_ref, w2_ref, b2_ref,
              w3_ref, b3_ref, o_ref):
    h = jnp.maximum(_f32dot(hv_ref[...], w1_ref[...]) + b1_ref[...], 0.0)
    h = jnp.maximum(_f32dot(h, w2_ref[...]) + b2_ref[...], 0.0)
    o_ref[...] = fr_ref[...] + _f32dot(h, w3_ref[...]) + b3_ref[...]


def _rows_call(body, grid, in_arrays, in_blocked_d, out_shapes, out_d, blk):
    in_specs = []
    for a, d in zip(in_arrays, in_blocked_d):
        if d is None:
            in_specs.append(_fullspec(a.shape))
        elif isinstance(d, tuple):
            in_specs.append(pl.BlockSpec((d[0], blk, d[1]),
                                         lambda i: (0, i, 0)))
        else:
            in_specs.append(_rowspec(blk, d))
    out_specs = [_rowspec(blk, d) for d in out_d]
    out_shape = [jax.ShapeDtypeStruct(s, jnp.float32) for s in out_shapes]
    if len(out_shape) == 1:
        out_shape, out_specs = out_shape[0], out_specs[0]
    return pl.pallas_call(
        body, grid=(grid,), in_specs=in_specs, out_specs=out_specs,
        out_shape=out_shape)(*in_arrays)


def _sc_gather_body(p_hbm, q_hbm, src_hbm, dst_hbm, r_hbm,
                    sidx, didx, pbuf, qbuf, sem1, sem2):
    cid = lax.axis_index("c")
    sid = lax.axis_index("s")
    wid = sid * _NC + cid
    nit = _JFULL + (wid < _REM).astype(jnp.int32)

    def body(j, carry):
        off = (j * _NW + wid) * _CH
        pltpu.sync_copy(src_hbm.at[pl.ds(off, _CH)], sidx)
        pltpu.sync_copy(dst_hbm.at[pl.ds(off, _CH)], didx)
        cp1 = pltpu.async_copy(p_hbm.at[sidx], pbuf, sem1)
        cp2 = pltpu.async_copy(q_hbm.at[didx], qbuf, sem2)
        cp1.wait()
        cp2.wait()

        def addrow(rr, c2):
            for cc in range(_H // 16):
                s = pbuf[rr, pl.ds(cc * 16, 16)] + qbuf[rr, pl.ds(cc * 16, 16)]
                pbuf[rr, pl.ds(cc * 16, 16)] = s
            return c2

        lax.fori_loop(0, _CH, addrow, 0)
        pltpu.sync_copy(pbuf, r_hbm.at[pl.ds(off, _CH)])
        return carry

    lax.fori_loop(0, nit, body, 0)


@functools.cache
def _sc_gather_fn():
    return pl.kernel(
        _sc_gather_body,
        out_type=jax.ShapeDtypeStruct((_E, _H), jnp.float32),
        mesh=_sc_mesh(),
        scratch_types=[
            pltpu.VMEM((_CH,), jnp.int32),
            pltpu.VMEM((_CH,), jnp.int32),
            pltpu.VMEM((_CH, _H), jnp.float32),
            pltpu.VMEM((_CH, _H), jnp.float32),
            pltpu.SemaphoreType.DMA,
            pltpu.SemaphoreType.DMA,
        ])


def _sc_gather(p, q, src, dst):
    return _sc_gather_fn()(p, q, src, dst)


def _sc_scatter_body(e_hbm, dst_hbm, z_hbm, out_hbm, didx, ebuf, acc):
    cid = lax.axis_index("c")
    sid = lax.axis_index("s")
    wid = sid * _NC + cid
    pltpu.sync_copy(z_hbm.at[pl.ds(sid * _NPT, _NPT)],
                    acc.at[pl.ds(sid * _NPT, _NPT)])

    @pl.when(sid == _NS - 1)
    def _():
        pltpu.sync_copy(z_hbm.at[pl.ds(_NS * _NPT, _NREM)],
                        acc.at[pl.ds(_NS * _NPT, _NREM)])

    plsc.subcore_barrier()
    nit = _JFULL + (wid < _REM).astype(jnp.int32)

    def body(j, carry):
        off = (j * _NW + wid) * _CH
        pltpu.sync_copy(dst_hbm.at[pl.ds(off, _CH)], didx)
        pltpu.sync_copy(e_hbm.at[pl.ds(off, _CH)], ebuf)
        pltpu.sync_copy(ebuf, acc.at[didx], add=True)
        return carry

    lax.fori_loop(0, nit, body, 0)
    plsc.subcore_barrier()
    pltpu.sync_copy(acc.at[pl.ds(sid * _NPT, _NPT)],
                    out_hbm.at[cid].at[pl.ds(sid * _NPT, _NPT)])

    @pl.when(sid == _NS - 1)
    def _():
        pltpu.sync_copy(acc.at[pl.ds(_NS * _NPT, _NREM)],
                        out_hbm.at[cid].at[pl.ds(_NS * _NPT, _NREM)])


@functools.cache
def _sc_scatter_fn():
    return pl.kernel(
        _sc_scatter_body,
        out_type=jax.ShapeDtypeStruct((_NC, _N, _H), jnp.float32),
        mesh=_sc_mesh(),
        scratch_types=[
            pltpu.VMEM((_CH,), jnp.int32),
            pltpu.VMEM((_CH, _H), jnp.float32),
            pltpu.VMEM_SHARED((_N, _H), jnp.float32),
        ])


def _sc_scatter(e_new, dst, zeros_nh):
    return _sc_scatter_fn()(e_new, dst, zeros_nh)


def kernel(graph_x, edge_index, edge_attr, velocity_sequence_noise,
           enc_node, enc_edge, mp_edge, mp_node, dec, norm_stats):
    del velocity_sequence_noise
    node_mean, node_std, edge_mean, edge_std, out_mean, out_std = norm_stats
    f32 = jnp.float32
    r1 = lambda a: a.reshape(1, -1).astype(f32)

    nw1 = enc_node[0] / node_std[:, None]
    nb1 = r1(enc_node[1] - (node_mean / node_std) @ enc_node[0])
    ew1 = enc_edge[0] / edge_std[:, None]
    eb1 = r1(enc_edge[1] - (edge_mean / edge_std) @ enc_edge[0])

    src = edge_index[0].astype(jnp.int32)
    dst = edge_index[1].astype(jnp.int32)
    frames = graph_x[:, 1:3]
    zeros_nh = jnp.zeros((_N, _H), f32)

    h_v = _rows_call(
        _enc_node_body, _N // _NB,
        [graph_x, nw1, nb1, enc_node[2], r1(enc_node[3]), enc_node[4],
         r1(enc_node[5]), r1(enc_node[6]), r1(enc_node[7])],
        [3] + [None] * 8, [(_N, _H)], [_H], _NB)

    h_e = _rows_call(
        _enc_edge_body, _E // _EB,
        [edge_attr, ew1, eb1, enc_edge[2], r1(enc_edge[3]), enc_edge[4],
         r1(enc_edge[5]), r1(enc_edge[6]), r1(enc_edge[7])],
        [3] + [None] * 8, [(_E, _H)], [_H], _EB)

    for i in range(_MP):
        we1, wb1, we2, wb2, we3, wb3, wg, wbe = mp_edge[i]
        w1e, w1s, w1d = we1[:_H], we1[_H:2 * _H], we1[2 * _H:]
        p, q = _rows_call(
            _pq_body, _N // _NB, [h_v, w1s, w1d], [_H, None, None],
            [(_N, _H), (_N, _H)], [_H, _H], _NB)

        r = _sc_gather(p, q, src, dst)

        e_new = _rows_call(
            _edge_mlp_body, _E // _EB,
            [h_e, r, w1e, r1(wb1), we2, r1(wb2), we3, r1(wb3), r1(wg),
             r1(wbe)],
            [_H, _H] + [None] * 8, [(_E, _H)], [_H], _EB)

        part = _sc_scatter(e_new, dst, zeros_nh)

        wn1, nb1_, wn2, nb2_, wn3, nb3_, ng_, nbe_ = mp_node[i]
        wv, wa = wn1[:_H], wn1[_H:]
        h_v = _rows_call(
            _node_mlp_body, _N // _NB,
            [h_v, part, wv, wa, r1(nb1_), wn2, r1(nb2_), wn3, r1(nb3_),
             r1(ng_), r1(nbe_)],
            [_H, (_NC, _H)] + [None] * 9, [(_N, _H)], [_H], _NB)
        h_e = e_new

    d1, db1, d2, db2, d3, db3 = dec
    d3f = d3 * out_std[None, :]
    db3f = r1(db3 * out_std + out_mean)
    out = _rows_call(
        _dec_body, _N // _NB,
        [h_v, frames, d1, r1(db1), d2, r1(db2), d3f, db3f],
        [_H, 2] + [None] * 6, [(_N, 2)], [2], _NB)
    return out
